# baseline (speedup 1.0000x reference)
import functools

import jax
import jax.numpy as jnp
from jax.experimental import pallas as pl
from jax.experimental.pallas import tpu as pltpu

_SELU_SCALE = 1.0507009873554805
_SELU_ALPHA = 1.6732632423543772

_E = 1600000
_N = 100000
_G = 128

_EROWS = 3136
_ECOLS = 512
_EPAD = _EROWS * _ECOLS
_EBLK = 64

_NPAD = 100096
_NBLK = 5888


def _act(x):
    return jnp.where(x > 0, x, _SELU_ALPHA * (jnp.exp(x) - 1.0))


def _selu(x):
    return _SELU_SCALE * _act(x)


def _edge_body(r_ref, s_ref, e_ref, p_ref, o_ref):
    r = r_ref[...]
    s = s_ref[...]
    e = e_ref[...]

    def p(i):
        return p_ref[0, i]

    h = [_act(r * p(k) + s * p(10 + k) + p(20 + k)) for k in range(10)]
    c = []
    for j in range(5):
        acc = h[0] * p(30 + j)
        for k in range(1, 10):
            acc = acc + h[k] * p(30 + 5 * k + j)
        c.append(acc + p(80 + j))
    for _ in range(3):
        u = [_act(e * p(85 + j) + c[j]) for j in range(5)]
        e = u[0] * p(90)
        for j in range(1, 5):
            e = e + u[j] * p(90 + j)
        e = e + p(95)
    o_ref[...] = e


@jax.jit
def _edge_stage(r2, s2, e2, params):
    grid = _EROWS // _EBLK
    blk = pl.BlockSpec((_EBLK, _ECOLS), lambda i: (i, 0))
    return pl.pallas_call(
        _edge_body,
        grid=(grid,),
        in_specs=[blk, blk, blk,
                  pl.BlockSpec(memory_space=pltpu.SMEM)],
        out_specs=blk,
        out_shape=jax.ShapeDtypeStruct((_EROWS, _ECOLS), jnp.float32),
    )(r2, s2, e2, params)


def _node_body(a_ref, n_ref, g_ref,
               pe_w1_ref, pe_b1_ref, pe_w2_ref, pe_b2_ref,
               un_w1_ref, un_b1_ref, un_w2_ref, un_b2_ref,
               pr_w1_ref, pr_b1_ref, pr_w2_ref, pr_b2_ref,
               o_ref, acc_ref):
    step = pl.program_id(0)
    a = a_ref[...]
    nz = n_ref[...]
    gid = g_ref[...]

    h2e = _selu(a @ pe_w1_ref[...] + pe_b1_ref[...]) @ pe_w2_ref[...] \
        + pe_b2_ref[...]
    h2n = nz
    for _ in range(3):
        t = jnp.concatenate([h2n, h2e], axis=1)
        h2n = _selu(t @ un_w1_ref[...] + un_b1_ref[...]) @ un_w2_ref[...] \
            + un_b2_ref[...]
    feats = jnp.concatenate([h2n, h2e], axis=1)

    iota = jax.lax.broadcasted_iota(jnp.int32, (_G, _NBLK), 0)
    oh = (iota == gid).astype(jnp.float32)
    contrib = jnp.dot(oh, feats, preferred_element_type=jnp.float32)

    @pl.when(step == 0)
    def _():
        acc_ref[...] = contrib

    @pl.when(step > 0)
    def _():
        acc_ref[...] = acc_ref[...] + contrib

    @pl.when(step == pl.num_programs(0) - 1)
    def _():
        t3 = acc_ref[...]
        out = _selu(t3 @ pr_w1_ref[...] + pr_b1_ref[...]) @ pr_w2_ref[...] \
            + pr_b2_ref[...]
        m = jnp.max(out, axis=-1, keepdims=True)
        ex = jnp.exp(out - m)
        o_ref[...] = ex / jnp.sum(ex, axis=-1, keepdims=True)


@jax.jit
def _node_stage(eonP, nodesP, gidP, pe_w1, pe_b1, pe_w2, pe_b2,
                un_w1, un_b1, un_w2, un_b2, pr_w1, pr_b1, pr_w2, pr_b2):
    grid = _NPAD // _NBLK
    col = pl.BlockSpec((_NBLK, 1), lambda i: (i, 0))
    row = pl.BlockSpec((1, _NBLK), lambda i: (0, i))
    wspecs = [pl.BlockSpec(w.shape, lambda i: (0, 0))
              for w in (pe_w1, pe_b1, pe_w2, pe_b2,
                        un_w1, un_b1, un_w2, un_b2,
                        pr_w1, pr_b1, pr_w2, pr_b2)]
    return pl.pallas_call(
        _node_body,
        grid=(grid,),
        in_specs=[col, col, row] + wspecs,
        out_specs=pl.BlockSpec((_G, 10), lambda i: (0, 0)),
        out_shape=jax.ShapeDtypeStruct((_G, 10), jnp.float32),
        scratch_shapes=[pltpu.VMEM((_G, 11), jnp.float32)],
    )(eonP, nodesP, gidP, pe_w1, pe_b1, pe_w2, pe_b2,
      un_w1, un_b1, un_w2, un_b2, pr_w1, pr_b1, pr_w2, pr_b2)


def kernel(nodes, edges, senders, receivers, graph_ids,
           pn_w1, pn_b1, pn_w2, pn_b2,
           ue_w1, ue_b1, ue_w2, ue_b2,
           pe_w1, pe_b1, pe_w2, pe_b2,
           un_w1, un_b1, un_w2, un_b2,
           pr_w1, pr_b1, pr_w2, pr_b2):
    nflat = nodes[:, 0]

    r = jnp.take(nflat, receivers)
    s = jnp.take(nflat, senders)

    w2c = _SELU_SCALE * (pn_w2 @ ue_w1[1:, :])
    bc = pn_b2 @ ue_w1[1:, :] + ue_b1
    params = jnp.concatenate([
        pn_w1[0], pn_w1[1], pn_b1, w2c.reshape(-1), bc,
        ue_w1[0], _SELU_SCALE * ue_w2[:, 0], ue_b2,
    ]).reshape(1, -1)
    epad = _EPAD - _E
    h1e = _edge_stage(jnp.pad(r, (0, epad)).reshape(_EROWS, _ECOLS),
                      jnp.pad(s, (0, epad)).reshape(_EROWS, _ECOLS),
                      jnp.pad(edges[:, 0], (0, epad)).reshape(_EROWS, _ECOLS),
                      params).reshape(_EPAD)[:_E]

    eon = jax.ops.segment_sum(h1e, receivers, num_segments=_N)

    pad = _NPAD - _N
    eonP = jnp.pad(eon, (0, pad))[:, None]
    nodesP = jnp.pad(nflat, (0, pad))[:, None]
    gidP = jnp.pad(graph_ids, (0, pad), constant_values=_G)[None, :]
    return _node_stage(
        eonP, nodesP, gidP,
        pe_w1, pe_b1.reshape(1, -1), pe_w2, pe_b2.reshape(1, -1),
        un_w1, un_b1.reshape(1, -1), un_w2, un_b2.reshape(1, -1),
        pr_w1, pr_b1.reshape(1, -1), pr_w2, pr_b2.reshape(1, -1))


# speedup vs baseline: 56.0502x; 56.0502x over previous
import functools

import jax
import jax.numpy as jnp
from jax import lax
from jax.experimental import pallas as pl
from jax.experimental.pallas import tpu as pltpu
from jax.experimental.pallas import tpu_sc as plsc

_SELU_SCALE = 1.0507009873554805
_SELU_ALPHA = 1.6732632423543772

_E = 1600000
_N = 100000
_G = 128

_EROWS = 3136
_ECOLS = 512
_EPAD = _EROWS * _ECOLS
_EBLK = 64

_NPAD = 100096
_NBLK = 5888

_NC = 2
_NS = 16
_NW = _NC * _NS
_EPW = _E // _NW
_GC = 2000
_GNC = _EPW // _GC
_ER2 = _EPAD // 128
_RPW = _ER2 // _NW
_SCR = 56
_SNC = _RPW // _SCR
_NACC = 100352
_NTILE = _NACC // _NS


def _gather_sc(nflat, senders, receivers):
    mesh = plsc.VectorSubcoreMesh(core_axis_name="c", subcore_axis_name="s")

    @functools.partial(
        pl.kernel, mesh=mesh,
        out_type=[jax.ShapeDtypeStruct((_E,), jnp.float32),
                  jax.ShapeDtypeStruct((_E,), jnp.float32)],
        scratch_types=[pltpu.VMEM((_N,), jnp.float32),
                       pltpu.VMEM((_GC,), jnp.int32),
                       pltpu.VMEM((_GC,), jnp.int32),
                       pltpu.VMEM((_GC,), jnp.float32),
                       pltpu.VMEM((_GC,), jnp.float32)],
        compiler_params=pltpu.CompilerParams(needs_layout_passes=False),
    )
    def k(table_hbm, snd_hbm, rcv_hbm, r_hbm, s_hbm,
          table_v, idxs_v, idxr_v, outs_v, outr_v):
        c = lax.axis_index("c")
        s = lax.axis_index("s")
        wid = c * _NS + s
        pltpu.sync_copy(table_hbm, table_v)

        def chunk(kk, carry):
            base = pl.multiple_of(wid * _EPW + kk * _GC, 8)
            pltpu.sync_copy(snd_hbm.at[pl.ds(base, _GC)], idxs_v)
            pltpu.sync_copy(rcv_hbm.at[pl.ds(base, _GC)], idxr_v)

            def gloop(i, carry2):
                off = pl.multiple_of(i * 16, 8)
                vi_s = idxs_v[pl.ds(off, 16)]
                vi_r = idxr_v[pl.ds(off, 16)]
                outs_v[pl.ds(off, 16)] = plsc.load_gather(table_v, [vi_s])
                outr_v[pl.ds(off, 16)] = plsc.load_gather(table_v, [vi_r])
                return carry2

            lax.fori_loop(0, _GC // 16, gloop, 0)
            pltpu.sync_copy(outs_v, s_hbm.at[pl.ds(base, _GC)])
            pltpu.sync_copy(outr_v, r_hbm.at[pl.ds(base, _GC)])
            return carry

        lax.fori_loop(0, _GNC, chunk, 0)

    return k(nflat, senders, receivers)


def _scatter_sc(recv2, val2, zeros):
    mesh = plsc.VectorSubcoreMesh(core_axis_name="c", subcore_axis_name="s")

    @functools.partial(
        pl.kernel, mesh=mesh,
        out_type=jax.ShapeDtypeStruct((_NC * _NACC,), jnp.float32),
        scratch_types=[pltpu.VMEM_SHARED((_NACC,), jnp.float32),
                       pltpu.VMEM((_SCR, 128), jnp.int32),
                       pltpu.VMEM((_SCR, 128), jnp.float32)],
        compiler_params=pltpu.CompilerParams(needs_layout_passes=False),
    )
    def k(recv_hbm, val_hbm, z_hbm, out_hbm, acc_sh, idx_v, val_v):
        c = lax.axis_index("c")
        s = lax.axis_index("s")
        wid = c * _NS + s
        toff = pl.multiple_of(s * _NTILE, 8)
        pltpu.sync_copy(z_hbm.at[pl.ds(toff, _NTILE)],
                        acc_sh.at[pl.ds(toff, _NTILE)])
        plsc.subcore_barrier()

        def chunk(kk, carry):
            rbase = pl.multiple_of(wid * _RPW + kk * _SCR, 8)
            pltpu.sync_copy(recv_hbm.at[pl.ds(rbase, _SCR)], idx_v)
            pltpu.sync_copy(val_hbm.at[pl.ds(rbase, _SCR)], val_v)
            for j in range(_SCR):
                pltpu.sync_copy(val_v.at[j], acc_sh.at[idx_v.at[j]], add=True)
            return carry

        lax.fori_loop(0, _SNC, chunk, 0)
        plsc.subcore_barrier()
        obase = pl.multiple_of(c * _NACC + s * _NTILE, 8)
        pltpu.sync_copy(acc_sh.at[pl.ds(toff, _NTILE)],
                        out_hbm.at[pl.ds(obase, _NTILE)])

    return k(recv2, val2, zeros)


def _act(x):
    return jnp.where(x > 0, x, _SELU_ALPHA * (jnp.exp(x) - 1.0))


def _selu(x):
    return _SELU_SCALE * _act(x)


def _edge_body(r_ref, s_ref, e_ref, p_ref, o_ref):
    r = r_ref[...]
    s = s_ref[...]
    e = e_ref[...]

    def p(i):
        return p_ref[0, i]

    h = [_act(r * p(k) + s * p(10 + k) + p(20 + k)) for k in range(10)]
    c = []
    for j in range(5):
        acc = h[0] * p(30 + j)
        for k in range(1, 10):
            acc = acc + h[k] * p(30 + 5 * k + j)
        c.append(acc + p(80 + j))
    for _ in range(3):
        u = [_act(e * p(85 + j) + c[j]) for j in range(5)]
        e = u[0] * p(90)
        for j in range(1, 5):
            e = e + u[j] * p(90 + j)
        e = e + p(95)
    o_ref[...] = e


@jax.jit
def _edge_stage(r2, s2, e2, params):
    grid = _EROWS // _EBLK
    blk = pl.BlockSpec((_EBLK, _ECOLS), lambda i: (i, 0))
    return pl.pallas_call(
        _edge_body,
        grid=(grid,),
        in_specs=[blk, blk, blk,
                  pl.BlockSpec(memory_space=pltpu.SMEM)],
        out_specs=blk,
        out_shape=jax.ShapeDtypeStruct((_EROWS, _ECOLS), jnp.float32),
    )(r2, s2, e2, params)


def _node_body(a0_ref, a1_ref, n_ref, g_ref,
               pe_w1_ref, pe_b1_ref, pe_w2_ref, pe_b2_ref,
               un_w1_ref, un_b1_ref, un_w2_ref, un_b2_ref,
               pr_w1_ref, pr_b1_ref, pr_w2_ref, pr_b2_ref,
               o_ref, acc_ref):
    step = pl.program_id(0)
    a = a0_ref[...] + a1_ref[...]
    nz = n_ref[...]
    gid = g_ref[...]

    h2e = _selu(a @ pe_w1_ref[...] + pe_b1_ref[...]) @ pe_w2_ref[...] \
        + pe_b2_ref[...]
    h2n = nz
    for _ in range(3):
        t = jnp.concatenate([h2n, h2e], axis=1)
        h2n = _selu(t @ un_w1_ref[...] + un_b1_ref[...]) @ un_w2_ref[...] \
            + un_b2_ref[...]
    feats = jnp.concatenate([h2n, h2e], axis=1)

    iota = jax.lax.broadcasted_iota(jnp.int32, (_G, _NBLK), 0)
    oh = (iota == gid).astype(jnp.float32)
    contrib = jnp.dot(oh, feats, preferred_element_type=jnp.float32)

    @pl.when(step == 0)
    def _():
        acc_ref[...] = contrib

    @pl.when(step > 0)
    def _():
        acc_ref[...] = acc_ref[...] + contrib

    @pl.when(step == pl.num_programs(0) - 1)
    def _():
        t3 = acc_ref[...]
        out = _selu(t3 @ pr_w1_ref[...] + pr_b1_ref[...]) @ pr_w2_ref[...] \
            + pr_b2_ref[...]
        m = jnp.max(out, axis=-1, keepdims=True)
        ex = jnp.exp(out - m)
        o_ref[...] = ex / jnp.sum(ex, axis=-1, keepdims=True)


@jax.jit
def _node_stage(eon0P, eon1P, nodesP, gidP, pe_w1, pe_b1, pe_w2, pe_b2,
                un_w1, un_b1, un_w2, un_b2, pr_w1, pr_b1, pr_w2, pr_b2):
    grid = _NPAD // _NBLK
    col = pl.BlockSpec((_NBLK, 1), lambda i: (i, 0))
    row = pl.BlockSpec((1, _NBLK), lambda i: (0, i))
    wspecs = [pl.BlockSpec(w.shape, lambda i: (0, 0))
              for w in (pe_w1, pe_b1, pe_w2, pe_b2,
                        un_w1, un_b1, un_w2, un_b2,
                        pr_w1, pr_b1, pr_w2, pr_b2)]
    return pl.pallas_call(
        _node_body,
        grid=(grid,),
        in_specs=[col, col, col, row] + wspecs,
        out_specs=pl.BlockSpec((_G, 10), lambda i: (0, 0)),
        out_shape=jax.ShapeDtypeStruct((_G, 10), jnp.float32),
        scratch_shapes=[pltpu.VMEM((_G, 11), jnp.float32)],
    )(eon0P, eon1P, nodesP, gidP, pe_w1, pe_b1, pe_w2, pe_b2,
      un_w1, un_b1, un_w2, un_b2, pr_w1, pr_b1, pr_w2, pr_b2)


def kernel(nodes, edges, senders, receivers, graph_ids,
           pn_w1, pn_b1, pn_w2, pn_b2,
           ue_w1, ue_b1, ue_w2, ue_b2,
           pe_w1, pe_b1, pe_w2, pe_b2,
           un_w1, un_b1, un_w2, un_b2,
           pr_w1, pr_b1, pr_w2, pr_b2):
    nflat = nodes[:, 0]

    r, s = _gather_sc(nflat, senders, receivers)

    w2c = _SELU_SCALE * (pn_w2 @ ue_w1[1:, :])
    bc = pn_b2 @ ue_w1[1:, :] + ue_b1
    params = jnp.concatenate([
        pn_w1[0], pn_w1[1], pn_b1, w2c.reshape(-1), bc,
        ue_w1[0], _SELU_SCALE * ue_w2[:, 0], ue_b2,
    ]).reshape(1, -1)
    epad = _EPAD - _E
    h1e = _edge_stage(jnp.pad(r, (0, epad)).reshape(_EROWS, _ECOLS),
                      jnp.pad(s, (0, epad)).reshape(_EROWS, _ECOLS),
                      jnp.pad(edges[:, 0], (0, epad)).reshape(_EROWS, _ECOLS),
                      params)

    pad_idx = _NPAD + (jnp.arange(epad, dtype=jnp.int32) % (_NACC - _NPAD))
    recv2 = jnp.concatenate([receivers, pad_idx]).reshape(_ER2, 128)
    val2 = h1e.reshape(_ER2, 128)
    zeros = jnp.zeros((_NACC,), jnp.float32)
    acc2 = _scatter_sc(recv2, val2, zeros)

    pad = _NPAD - _N
    eon0P = acc2[:_NPAD][:, None]
    eon1P = acc2[_NACC:_NACC + _NPAD][:, None]
    nodesP = jnp.pad(nflat, (0, pad))[:, None]
    gidP = jnp.pad(graph_ids, (0, pad), constant_values=_G)[None, :]
    return _node_stage(
        eon0P, eon1P, nodesP, gidP,
        pe_w1, pe_b1.reshape(1, -1), pe_w2, pe_b2.reshape(1, -1),
        un_w1, un_b1.reshape(1, -1), un_w2, un_b2.reshape(1, -1),
        pr_w1, pr_b1.reshape(1, -1), pr_w2, pr_b2.reshape(1, -1))
